# Initial kernel scaffold; baseline (speedup 1.0000x reference)
import jax
import jax.numpy as jnp
from jax.experimental import pallas as pl


def kernel(x, edge_index, edge_attr, params):
    raise NotImplementedError("write your pallas kernel here")


# speedup vs baseline: 2.3572x; 2.3572x over previous
import functools

import jax
import jax.numpy as jnp
from jax import lax
from jax.experimental import pallas as pl
from jax.experimental.pallas import tpu as pltpu
from jax.experimental.pallas import tpu_sc as plsc

_F32 = jnp.float32


def _sc_gather_rows(table, idx):
    n, d = table.shape
    e = idx.shape[0]
    nw = 32
    per = e // nw
    ch = 2000
    assert e % nw == 0 and per % ch == 0

    mesh = plsc.VectorSubcoreMesh(core_axis_name="c", subcore_axis_name="s")

    @functools.partial(
        pl.kernel,
        mesh=mesh,
        out_type=jax.ShapeDtypeStruct((e, d), _F32),
        scratch_types=[
            pltpu.VMEM((ch,), jnp.int32),
            pltpu.VMEM((ch, d), _F32),
            pltpu.SemaphoreType.DMA,
        ],
        compiler_params=pltpu.CompilerParams(use_tc_tiling_on_sc=False),
    )
    def k(table_hbm, idx_hbm, out_hbm, idx_v, rows_v, sem):
        wid = lax.axis_index("s") * 2 + lax.axis_index("c")
        base = wid * per

        def body(i, carry):
            off = base + i * ch
            pltpu.sync_copy(idx_hbm.at[pl.ds(off, ch)], idx_v)
            pltpu.async_copy(table_hbm.at[idx_v], rows_v, sem).wait()
            pltpu.sync_copy(rows_v, out_hbm.at[pl.ds(off, ch)])
            return carry

        lax.fori_loop(0, per // ch, body, 0)

    return k(table, idx)


def _sc_scatter_add(msg, dst, f_prev):
    e, d = msg.shape
    n = f_prev.shape[0]
    nt = 16
    per = e // nt
    ch = 2000
    assert e % nt == 0 and per % ch == 0

    mesh = plsc.VectorSubcoreMesh(
        core_axis_name="c", subcore_axis_name="s", num_cores=1
    )

    @functools.partial(
        pl.kernel,
        mesh=mesh,
        out_type=jax.ShapeDtypeStruct((n, d), _F32),
        scratch_types=[
            pltpu.VMEM((ch,), jnp.int32),
            pltpu.VMEM((ch, d), _F32),
            pltpu.VMEM_SHARED((n, d), _F32),
            pltpu.SemaphoreType.DMA,
        ],
        compiler_params=pltpu.CompilerParams(use_tc_tiling_on_sc=False),
    )
    def k(msg_hbm, dst_hbm, fprev_hbm, out_hbm, idx_v, upd_v, acc_sh, sem):
        sid = lax.axis_index("s")

        @pl.when(sid == 0)
        def _():
            pltpu.sync_copy(fprev_hbm, acc_sh)

        plsc.subcore_barrier()
        base = sid * per

        def body(i, carry):
            off = base + i * ch
            pltpu.sync_copy(dst_hbm.at[pl.ds(off, ch)], idx_v)
            pltpu.sync_copy(msg_hbm.at[pl.ds(off, ch)], upd_v)
            pltpu.sync_copy(upd_v, acc_sh.at[idx_v], add=True)
            return carry

        lax.fori_loop(0, per // ch, body, 0)
        plsc.subcore_barrier()

        @pl.when(sid == 0)
        def _():
            pltpu.sync_copy(acc_sh, out_hbm)

    return k(msg, dst, f_prev)


def _prelu(h, a):
    return jnp.where(h >= 0.0, h, a * h)


def _mlp_body(first, vi_ref, vj_ref, fj_ref, ea_ref, dirc_ref, wa_ref, wb_ref,
              wf_ref, wc_ref, b0_ref, w1_ref, w2_ref, w3_ref, wl8_ref,
              alpha_ref, out_ref):
    vi = vi_ref[...]
    vj = vj_ref[...]
    if first:
        h = jnp.dot(vi * vj, wa_ref[...]) + jnp.dot(ea_ref[...], wc_ref[...])
    else:
        h = (
            jnp.dot(vi, wa_ref[...])
            + jnp.dot(vj, wb_ref[...])
            + jnp.dot(fj_ref[...], wf_ref[...])
            + jnp.dot(ea_ref[...], wc_ref[...])
        )
    h = _prelu(h + b0_ref[...], alpha_ref[0, 0])
    h = _prelu(jnp.dot(h, w1_ref[...]), alpha_ref[0, 1])
    h = _prelu(jnp.dot(h, w2_ref[...]), alpha_ref[0, 2])
    h = _prelu(jnp.dot(h, w3_ref[...]), alpha_ref[0, 3])
    t = jnp.dot(h, wl8_ref[...]) + alpha_ref[0, 4]
    out_ref[...] = t * dirc_ref[...]


def _tc_mlp(first, vi, vj, fj, ea8, dirc, wa, wb, wf, wc, b0, ws, wl8, alphas):
    e = vi.shape[0]
    eb = 1280
    assert e % eb == 0
    grid = (e // eb,)

    edge_spec = pl.BlockSpec((eb, 8), lambda i: (i, 0))
    w_in_spec = pl.BlockSpec((8, 128), lambda i: (0, 0))
    w_hid_spec = pl.BlockSpec((128, 128), lambda i: (0, 0))
    specs = [
        edge_spec,
        edge_spec,
        edge_spec,
        edge_spec,
        edge_spec,
        w_in_spec,
        w_in_spec,
        w_in_spec,
        w_in_spec,
        pl.BlockSpec((1, 128), lambda i: (0, 0)),
        w_hid_spec,
        w_hid_spec,
        w_hid_spec,
        pl.BlockSpec((128, 8), lambda i: (0, 0)),
        pl.BlockSpec(memory_space=pltpu.MemorySpace.SMEM),
    ]

    return pl.pallas_call(
        functools.partial(_mlp_body, first),
        grid=grid,
        in_specs=specs,
        out_specs=edge_spec,
        out_shape=jax.ShapeDtypeStruct((e, 8), _F32),
    )(vi, vj, fj, ea8, dirc, wa, wb, wf, wc, b0, ws[0], ws[1], ws[2], wl8,
      alphas)


def _prep_weights(p, first):
    w0 = p["W0"]
    z = jnp.zeros((8, 128), _F32)
    if first:
        wa = z.at[0:3].set(w0[0:3])
        wb = z
        wf = z
        wc = z.at[0:4].set(w0[3:7])
    else:
        wa = z.at[0:3].set(w0[0:3])
        wb = z.at[0:3].set(w0[3:6])
        wf = z.at[0:4].set(w0[7:11])
        wc = z.at[3:4].set(w0[6:7])
    b0 = p["b0"].reshape(1, 128)
    wl8 = jnp.broadcast_to(p["Wl"], (128, 8))
    scal = jnp.stack(
        [
            p["a0"],
            p["alphas"][0],
            p["alphas"][1],
            p["alphas"][2],
            p["bl"][0],
            jnp.float32(0.0),
            jnp.float32(0.0),
            jnp.float32(0.0),
        ]
    ).reshape(1, 8)
    return wa, wb, wf, wc, b0, tuple(p["Ws"]), wl8, scal


def kernel(x, edge_index, edge_attr, params):
    n = x.shape[0]
    e = edge_index.shape[1]
    src = edge_index[0].astype(jnp.int32)
    dst = edge_index[1].astype(jnp.int32)

    v8 = jnp.pad(x.astype(_F32), ((0, 0), (0, 5)))
    ea8 = jnp.pad(edge_attr.astype(_F32), ((0, 0), (0, 1)))
    direction = edge_attr[:, 4:7].astype(_F32)
    dirc = jnp.concatenate(
        [direction, jnp.ones((e, 1), _F32), jnp.zeros((e, 4), _F32)], axis=1
    )

    vi = _sc_gather_rows(v8, dst)
    vj = _sc_gather_rows(v8, src)

    f = jnp.zeros((n, 8), _F32)
    fj = jnp.zeros((e, 8), _F32)
    for li, name in enumerate(("mlp0", "mlp1", "mlp2")):
        first = li == 0
        wa, wb, wf, wc, b0, ws, wl8, alphas = _prep_weights(
            params[name], first
        )
        if not first:
            fj = _sc_gather_rows(f, src)
        msg = _tc_mlp(first, vi, vj, fj, ea8, dirc, wa, wb, wf, wc, b0, ws,
                      wl8, alphas)
        f = _sc_scatter_add(msg, dst, f)

    return f[:, :3]


# speedup vs baseline: 2.7174x; 1.1528x over previous
import functools

import jax
import jax.numpy as jnp
from jax import lax
from jax.experimental import pallas as pl
from jax.experimental.pallas import tpu as pltpu
from jax.experimental.pallas import tpu_sc as plsc

_F32 = jnp.float32


def _sc_gather_rows(table, idx):
    n, d = table.shape
    e = idx.shape[0]
    nw = 32
    per = e // nw
    ch = 2000
    assert e % nw == 0 and per % ch == 0

    mesh = plsc.VectorSubcoreMesh(core_axis_name="c", subcore_axis_name="s")

    @functools.partial(
        pl.kernel,
        mesh=mesh,
        out_type=jax.ShapeDtypeStruct((e, d), _F32),
        scratch_types=[
            pltpu.VMEM((ch,), jnp.int32),
            pltpu.VMEM((ch, d), _F32),
            pltpu.SemaphoreType.DMA,
        ],
        compiler_params=pltpu.CompilerParams(use_tc_tiling_on_sc=False),
    )
    def k(table_hbm, idx_hbm, out_hbm, idx_v, rows_v, sem):
        wid = lax.axis_index("s") * 2 + lax.axis_index("c")
        base = wid * per

        def body(i, carry):
            off = base + i * ch
            pltpu.sync_copy(idx_hbm.at[pl.ds(off, ch)], idx_v)
            pltpu.async_copy(table_hbm.at[idx_v], rows_v, sem).wait()
            pltpu.sync_copy(rows_v, out_hbm.at[pl.ds(off, ch)])
            return carry

        lax.fori_loop(0, per // ch, body, 0)

    return k(table, idx)


def _sc_scatter_add(msg, dst, f_prev):
    e, d = msg.shape
    n = f_prev.shape[0]
    nt = 16
    per = e // nt
    ch = 2000
    assert e % nt == 0 and per % ch == 0

    mesh = plsc.VectorSubcoreMesh(
        core_axis_name="c", subcore_axis_name="s", num_cores=1
    )

    @functools.partial(
        pl.kernel,
        mesh=mesh,
        out_type=jax.ShapeDtypeStruct((n, d), _F32),
        scratch_types=[
            pltpu.VMEM((ch,), jnp.int32),
            pltpu.VMEM((ch, d), _F32),
            pltpu.VMEM_SHARED((n, d), _F32),
            pltpu.SemaphoreType.DMA,
        ],
        compiler_params=pltpu.CompilerParams(use_tc_tiling_on_sc=False),
    )
    def k(msg_hbm, dst_hbm, fprev_hbm, out_hbm, idx_v, upd_v, acc_sh, sem):
        sid = lax.axis_index("s")

        @pl.when(sid == 0)
        def _():
            pltpu.sync_copy(fprev_hbm, acc_sh)

        plsc.subcore_barrier()
        base = sid * per

        def body(i, carry):
            off = base + i * ch
            pltpu.sync_copy(dst_hbm.at[pl.ds(off, ch)], idx_v)
            pltpu.sync_copy(msg_hbm.at[pl.ds(off, ch)], upd_v)
            pltpu.sync_copy(upd_v, acc_sh.at[idx_v], add=True)
            return carry

        lax.fori_loop(0, per // ch, body, 0)
        plsc.subcore_barrier()

        @pl.when(sid == 0)
        def _():
            pltpu.sync_copy(acc_sh, out_hbm)

    return k(msg, dst, f_prev)


def _prelu(h, a):
    return jnp.where(h >= 0.0, h, a * h)


_HID_PREC = jax.lax.Precision.DEFAULT


def _mlp_body(first, vi_ref, vj_ref, *rest):
    if first:
        (ea_ref, dirc_ref, wa_ref, wc_ref, b0_ref, w1_ref, w2_ref, w3_ref,
         wl8_ref, alpha_ref, out_ref) = rest
        h = jnp.dot(vi_ref[...] * vj_ref[...], wa_ref[...]) + jnp.dot(
            ea_ref[...], wc_ref[...]
        )
    else:
        (fj_ref, ea_ref, dirc_ref, wa_ref, wb_ref, wf_ref, wc_ref, b0_ref,
         w1_ref, w2_ref, w3_ref, wl8_ref, alpha_ref, out_ref) = rest
        h = (
            jnp.dot(vi_ref[...], wa_ref[...])
            + jnp.dot(vj_ref[...], wb_ref[...])
            + jnp.dot(fj_ref[...], wf_ref[...])
            + jnp.dot(ea_ref[...], wc_ref[...])
        )
    h = _prelu(h + b0_ref[...], alpha_ref[0, 0])
    h = _prelu(jnp.dot(h, w1_ref[...], precision=_HID_PREC), alpha_ref[0, 1])
    h = _prelu(jnp.dot(h, w2_ref[...], precision=_HID_PREC), alpha_ref[0, 2])
    h = _prelu(jnp.dot(h, w3_ref[...], precision=_HID_PREC), alpha_ref[0, 3])
    t = jnp.dot(h, wl8_ref[...]) + alpha_ref[0, 4]
    out_ref[...] = t * dirc_ref[...]


def _tc_mlp(first, vi, vj, fj, ea8, dirc, wa, wb, wf, wc, b0, ws, wl8, alphas):
    e = vi.shape[0]
    eb = 2560
    assert e % eb == 0
    grid = (e // eb,)

    edge_spec = pl.BlockSpec((eb, 8), lambda i: (i, 0))
    w_in_spec = pl.BlockSpec((8, 128), lambda i: (0, 0))
    w_hid_spec = pl.BlockSpec((128, 128), lambda i: (0, 0))
    tail_specs = [
        pl.BlockSpec((1, 128), lambda i: (0, 0)),
        w_hid_spec,
        w_hid_spec,
        w_hid_spec,
        pl.BlockSpec((128, 8), lambda i: (0, 0)),
        pl.BlockSpec(memory_space=pltpu.MemorySpace.SMEM),
    ]
    if first:
        specs = [edge_spec] * 4 + [w_in_spec] * 2 + tail_specs
        args = (vi, vj, ea8, dirc, wa, wc, b0, ws[0], ws[1], ws[2], wl8,
                alphas)
    else:
        specs = [edge_spec] * 5 + [w_in_spec] * 4 + tail_specs
        args = (vi, vj, fj, ea8, dirc, wa, wb, wf, wc, b0, ws[0], ws[1],
                ws[2], wl8, alphas)

    return pl.pallas_call(
        functools.partial(_mlp_body, first),
        grid=grid,
        in_specs=specs,
        out_specs=edge_spec,
        out_shape=jax.ShapeDtypeStruct((e, 8), _F32),
    )(*args)


def _prep_weights(p, first):
    w0 = p["W0"]
    z = jnp.zeros((8, 128), _F32)
    if first:
        wa = z.at[0:3].set(w0[0:3])
        wb = z
        wf = z
        wc = z.at[0:4].set(w0[3:7])
    else:
        wa = z.at[0:3].set(w0[0:3])
        wb = z.at[0:3].set(w0[3:6])
        wf = z.at[0:4].set(w0[7:11])
        wc = z.at[3:4].set(w0[6:7])
    b0 = p["b0"].reshape(1, 128)
    wl8 = jnp.broadcast_to(p["Wl"], (128, 8))
    scal = jnp.stack(
        [
            p["a0"],
            p["alphas"][0],
            p["alphas"][1],
            p["alphas"][2],
            p["bl"][0],
            jnp.float32(0.0),
            jnp.float32(0.0),
            jnp.float32(0.0),
        ]
    ).reshape(1, 8)
    return wa, wb, wf, wc, b0, tuple(p["Ws"]), wl8, scal


def kernel(x, edge_index, edge_attr, params):
    n = x.shape[0]
    e = edge_index.shape[1]
    src = edge_index[0].astype(jnp.int32)
    dst = edge_index[1].astype(jnp.int32)

    v8 = jnp.pad(x.astype(_F32), ((0, 0), (0, 5)))
    ea8 = jnp.pad(edge_attr.astype(_F32), ((0, 0), (0, 1)))
    direction = edge_attr[:, 4:7].astype(_F32)
    dirc = jnp.concatenate(
        [direction, jnp.ones((e, 1), _F32), jnp.zeros((e, 4), _F32)], axis=1
    )

    vi = _sc_gather_rows(v8, dst)
    vj = _sc_gather_rows(v8, src)

    f = jnp.zeros((n, 8), _F32)
    fj = None
    for li, name in enumerate(("mlp0", "mlp1", "mlp2")):
        first = li == 0
        wa, wb, wf, wc, b0, ws, wl8, alphas = _prep_weights(
            params[name], first
        )
        if not first:
            fj = _sc_gather_rows(f, src)
        msg = _tc_mlp(first, vi, vj, fj, ea8, dirc, wa, wb, wf, wc, b0, ws,
                      wl8, alphas)
        f = _sc_scatter_add(msg, dst, f)

    return f[:, :3]


# speedup vs baseline: 5.3474x; 1.9678x over previous
import functools

import jax
import jax.numpy as jnp
from jax import lax
from jax.experimental import pallas as pl
from jax.experimental.pallas import tpu as pltpu
from jax.experimental.pallas import tpu_sc as plsc

_F32 = jnp.float32
_I32 = jnp.int32
_SC_PARAMS = pltpu.CompilerParams(
    use_tc_tiling_on_sc=False, needs_layout_passes=False
)


def _sc_gather_planes(table, idxs, n_feat):
    n, d = table.shape
    e = idxs[0].shape[0]
    nw = 32
    per = e // nw
    ch = 2000
    assert e % nw == 0 and per % ch == 0 and d == 8

    mesh = plsc.VectorSubcoreMesh(core_axis_name="c", subcore_axis_name="s")
    n_out = len(idxs)

    @functools.partial(
        pl.kernel,
        mesh=mesh,
        out_type=tuple(
            jax.ShapeDtypeStruct((8, e), _F32) for _ in range(n_out)
        ),
        scratch_types=[
            pltpu.VMEM((n, d), _F32),
            pltpu.VMEM((ch,), _I32),
            pltpu.VMEM((8, ch), _F32),
            pltpu.SemaphoreType.DMA,
        ],
        compiler_params=_SC_PARAMS,
    )
    def k(table_hbm, *rest):
        idx_hbms = rest[:n_out]
        out_hbms = rest[n_out:2 * n_out]
        tab_v, idx_v, planes_v, sem = rest[2 * n_out:]
        wid = lax.axis_index("s") * 2 + lax.axis_index("c")
        base = wid * per
        pltpu.sync_copy(table_hbm, tab_v)
        zeros16 = jnp.zeros((16,), _F32)

        def chunk(args):
            idx_hbm, out_hbm, i = args
            off = base + i * ch
            pltpu.sync_copy(idx_hbm.at[pl.ds(off, ch)], idx_v)

            def group(g, carry):
                nodes = idx_v[pl.ds(g * 16, 16)]
                for f in range(8):
                    if f < n_feat:
                        vals = plsc.load_gather(
                            tab_v, [nodes, jnp.full((16,), f, _I32)]
                        )
                    else:
                        vals = zeros16
                    planes_v[f, pl.ds(g * 16, 16)] = vals
                return carry

            lax.fori_loop(0, ch // 16, group, 0)
            pltpu.sync_copy(planes_v, out_hbm.at[:, pl.ds(off, ch)])

        for idx_hbm, out_hbm in zip(idx_hbms, out_hbms):
            def body(i, carry, idx_hbm=idx_hbm, out_hbm=out_hbm):
                chunk((idx_hbm, out_hbm, i))
                return carry

            lax.fori_loop(0, per // ch, body, 0)

    return k(table, *idxs)


def _sc_scatter_add(msg_t, dst, f_prev):
    d, e = msg_t.shape
    n = f_prev.shape[0]
    nt = 16
    per = e // nt
    ch = 2000
    assert e % nt == 0 and per % ch == 0 and d == 8

    mesh = plsc.VectorSubcoreMesh(
        core_axis_name="c", subcore_axis_name="s", num_cores=1
    )

    @functools.partial(
        pl.kernel,
        mesh=mesh,
        out_type=jax.ShapeDtypeStruct((n, d), _F32),
        scratch_types=[
            pltpu.VMEM((ch,), _I32),
            pltpu.VMEM((8, ch), _F32),
            pltpu.VMEM((ch, 8), _F32),
            pltpu.VMEM_SHARED((n, d), _F32),
            pltpu.SemaphoreType.DMA,
        ],
        compiler_params=_SC_PARAMS,
    )
    def k(msg_hbm, dst_hbm, fprev_hbm, out_hbm, idx_v, planes_v, rows_v,
          acc_sh, sem):
        sid = lax.axis_index("s")

        @pl.when(sid == 0)
        def _():
            pltpu.sync_copy(fprev_hbm, acc_sh)

        plsc.subcore_barrier()
        base = sid * per
        lanes = lax.iota(_I32, 16)

        def body(i, carry):
            off = base + i * ch
            pltpu.sync_copy(dst_hbm.at[pl.ds(off, ch)], idx_v)
            pltpu.sync_copy(msg_hbm.at[:, pl.ds(off, ch)], planes_v)

            def group(g, carry2):
                rows = g * 16 + lanes
                for f in range(8):
                    vals = planes_v[f, pl.ds(g * 16, 16)]
                    plsc.store_scatter(
                        rows_v, [rows, jnp.full((16,), f, _I32)], vals
                    )
                return carry2

            lax.fori_loop(0, ch // 16, group, 0)
            pltpu.sync_copy(rows_v, acc_sh.at[idx_v], add=True)
            return carry

        lax.fori_loop(0, per // ch, body, 0)
        plsc.subcore_barrier()

        @pl.when(sid == 0)
        def _():
            pltpu.sync_copy(acc_sh, out_hbm)

    return k(msg_t, dst, f_prev)


def _ea_body(ea_ref, eat_ref, dirc_ref):
    ea = ea_ref[...]
    eb = ea.shape[0]
    ea8 = jnp.concatenate([ea, jnp.zeros((eb, 1), _F32)], axis=1)
    eat = jnp.transpose(ea8)
    eat_ref[...] = eat
    dirc_ref[...] = jnp.concatenate(
        [eat[4:7, :], jnp.ones((1, eb), _F32), jnp.zeros((4, eb), _F32)],
        axis=0,
    )


def _tc_prep_edges(edge_attr):
    e = edge_attr.shape[0]
    eb = 2560
    assert e % eb == 0
    out_spec = pl.BlockSpec((8, eb), lambda i: (0, i))
    return pl.pallas_call(
        _ea_body,
        grid=(e // eb,),
        in_specs=[pl.BlockSpec((eb, 7), lambda i: (i, 0))],
        out_specs=(out_spec, out_spec),
        out_shape=(
            jax.ShapeDtypeStruct((8, e), _F32),
            jax.ShapeDtypeStruct((8, e), _F32),
        ),
    )(edge_attr)


def _prelu(h, a):
    return jnp.where(h >= 0.0, h, a * h)


def _mlp_body(first, vi_ref, vj_ref, *rest):
    if first:
        (ea_ref, dirc_ref, wa_ref, wc_ref, b0_ref, w1_ref, w2_ref, w3_ref,
         wl8_ref, alpha_ref, out_ref) = rest
        h = jnp.dot(wa_ref[...], vi_ref[...] * vj_ref[...]) + jnp.dot(
            wc_ref[...], ea_ref[...]
        )
    else:
        (fj_ref, ea_ref, dirc_ref, wa_ref, wb_ref, wf_ref, wc_ref, b0_ref,
         w1_ref, w2_ref, w3_ref, wl8_ref, alpha_ref, out_ref) = rest
        h = (
            jnp.dot(wa_ref[...], vi_ref[...])
            + jnp.dot(wb_ref[...], vj_ref[...])
            + jnp.dot(wf_ref[...], fj_ref[...])
            + jnp.dot(wc_ref[...], ea_ref[...])
        )
    h = _prelu(h + b0_ref[...], alpha_ref[0, 0])
    h = _prelu(jnp.dot(w1_ref[...], h), alpha_ref[0, 1])
    h = _prelu(jnp.dot(w2_ref[...], h), alpha_ref[0, 2])
    h = _prelu(jnp.dot(w3_ref[...], h), alpha_ref[0, 3])
    t = jnp.dot(wl8_ref[...], h) + alpha_ref[0, 4]
    out_ref[...] = t * dirc_ref[...]


def _tc_mlp(first, vi, vj, fj, ea_t, dirc_t, wa, wb, wf, wc, b0, ws, wl8,
            alphas):
    e = vi.shape[1]
    eb = 2560
    assert e % eb == 0
    grid = (e // eb,)

    edge_spec = pl.BlockSpec((8, eb), lambda i: (0, i))
    w_in_spec = pl.BlockSpec((128, 8), lambda i: (0, 0))
    w_hid_spec = pl.BlockSpec((128, 128), lambda i: (0, 0))
    tail_specs = [
        pl.BlockSpec((128, 1), lambda i: (0, 0)),
        w_hid_spec,
        w_hid_spec,
        w_hid_spec,
        pl.BlockSpec((8, 128), lambda i: (0, 0)),
        pl.BlockSpec(memory_space=pltpu.MemorySpace.SMEM),
    ]
    if first:
        specs = [edge_spec] * 4 + [w_in_spec] * 2 + tail_specs
        args = (vi, vj, ea_t, dirc_t, wa, wc, b0, ws[0], ws[1], ws[2], wl8,
                alphas)
    else:
        specs = [edge_spec] * 5 + [w_in_spec] * 4 + tail_specs
        args = (vi, vj, fj, ea_t, dirc_t, wa, wb, wf, wc, b0, ws[0], ws[1],
                ws[2], wl8, alphas)

    return pl.pallas_call(
        functools.partial(_mlp_body, first),
        grid=grid,
        in_specs=specs,
        out_specs=edge_spec,
        out_shape=jax.ShapeDtypeStruct((8, e), _F32),
    )(*args)


def _prep_weights(p, first):
    w0 = p["W0"]
    z = jnp.zeros((128, 8), _F32)
    if first:
        wa = z.at[:, 0:3].set(w0[0:3].T)
        wb = z
        wf = z
        wc = z.at[:, 0:4].set(w0[3:7].T)
    else:
        wa = z.at[:, 0:3].set(w0[0:3].T)
        wb = z.at[:, 0:3].set(w0[3:6].T)
        wf = z.at[:, 0:4].set(w0[7:11].T)
        wc = z.at[:, 3:4].set(w0[6:7].T)
    b0 = p["b0"].reshape(128, 1)
    ws = tuple(w.T for w in p["Ws"])
    wl8 = jnp.broadcast_to(p["Wl"].T, (8, 128))
    scal = jnp.stack(
        [
            p["a0"],
            p["alphas"][0],
            p["alphas"][1],
            p["alphas"][2],
            p["bl"][0],
            jnp.float32(0.0),
            jnp.float32(0.0),
            jnp.float32(0.0),
        ]
    ).reshape(1, 8)
    return wa, wb, wf, wc, b0, ws, wl8, scal


def kernel(x, edge_index, edge_attr, params):
    n = x.shape[0]
    e = edge_index.shape[1]
    src = edge_index[0].astype(_I32)
    dst = edge_index[1].astype(_I32)

    v8 = jnp.pad(x.astype(_F32), ((0, 0), (0, 5)))
    ea_t, dirc_t = _tc_prep_edges(edge_attr.astype(_F32))

    vi, vj = _sc_gather_planes(v8, (dst, src), 3)

    f = jnp.zeros((n, 8), _F32)
    fj = None
    for li, name in enumerate(("mlp0", "mlp1", "mlp2")):
        first = li == 0
        wa, wb, wf, wc, b0, ws, wl8, alphas = _prep_weights(
            params[name], first
        )
        if not first:
            (fj,) = _sc_gather_planes(f, (src,), 4)
        msg_t = _tc_mlp(first, vi, vj, fj, ea_t, dirc_t, wa, wb, wf, wc, b0,
                        ws, wl8, alphas)
        f = _sc_scatter_add(msg_t, dst, f)

    return f[:, :3]


# speedup vs baseline: 6.6519x; 1.2439x over previous
import functools

import jax
import jax.numpy as jnp
from jax import lax
from jax.experimental import pallas as pl
from jax.experimental.pallas import tpu as pltpu
from jax.experimental.pallas import tpu_sc as plsc

_F32 = jnp.float32
_I32 = jnp.int32
_SC_PARAMS = pltpu.CompilerParams(
    use_tc_tiling_on_sc=False, needs_layout_passes=False
)


def _sc_gather_planes(table, idxs, n_feat):
    n, d = table.shape
    e = idxs[0].shape[0]
    nw = 32
    per = e // nw
    ch = 2000
    assert e % nw == 0 and per % ch == 0 and d == 8

    mesh = plsc.VectorSubcoreMesh(core_axis_name="c", subcore_axis_name="s")
    n_out = len(idxs)

    @functools.partial(
        pl.kernel,
        mesh=mesh,
        out_type=tuple(
            jax.ShapeDtypeStruct((8, e), _F32) for _ in range(n_out)
        ),
        scratch_types=[
            pltpu.VMEM((n, d), _F32),
            pltpu.VMEM((ch,), _I32),
            pltpu.VMEM((8, ch), _F32),
            pltpu.SemaphoreType.DMA,
        ],
        compiler_params=_SC_PARAMS,
    )
    def k(table_hbm, *rest):
        idx_hbms = rest[:n_out]
        out_hbms = rest[n_out:2 * n_out]
        tab_v, idx_v, planes_v, sem = rest[2 * n_out:]
        wid = lax.axis_index("s") * 2 + lax.axis_index("c")
        base = wid * per
        pltpu.sync_copy(table_hbm, tab_v)
        zeros16 = jnp.zeros((16,), _F32)

        def chunk(args):
            idx_hbm, out_hbm, i = args
            off = base + i * ch
            pltpu.sync_copy(idx_hbm.at[pl.ds(off, ch)], idx_v)

            def group(g, carry):
                nodes = idx_v[pl.ds(g * 16, 16)]
                for f in range(8):
                    if f < n_feat:
                        vals = plsc.load_gather(
                            tab_v, [nodes, jnp.full((16,), f, _I32)]
                        )
                    else:
                        vals = zeros16
                    planes_v[f, pl.ds(g * 16, 16)] = vals
                return carry

            lax.fori_loop(0, ch // 16, group, 0)
            pltpu.sync_copy(planes_v, out_hbm.at[:, pl.ds(off, ch)])

        for idx_hbm, out_hbm in zip(idx_hbms, out_hbms):
            def body(i, carry, idx_hbm=idx_hbm, out_hbm=out_hbm):
                chunk((idx_hbm, out_hbm, i))
                return carry

            lax.fori_loop(0, per // ch, body, 0)

    return k(table, *idxs)


def _sc_scatter_add(msg_t, dst, f_prev):
    d, e = msg_t.shape
    n = f_prev.shape[0]
    nt = 16
    per = e // nt
    ch = 2000
    assert e % nt == 0 and per % ch == 0 and d == 8

    mesh = plsc.VectorSubcoreMesh(
        core_axis_name="c", subcore_axis_name="s", num_cores=1
    )

    nch = per // ch
    assert nch % 2 == 0

    @functools.partial(
        pl.kernel,
        mesh=mesh,
        out_type=jax.ShapeDtypeStruct((n, d), _F32),
        scratch_types=[
            pltpu.VMEM((2, ch), _I32),
            pltpu.VMEM((2, 8, ch), _F32),
            pltpu.VMEM((ch, 8), _F32),
            pltpu.VMEM_SHARED((n, d), _F32),
            pltpu.SemaphoreType.DMA,
            pltpu.SemaphoreType.DMA,
            pltpu.SemaphoreType.DMA,
            pltpu.SemaphoreType.DMA,
        ],
        compiler_params=_SC_PARAMS,
    )
    def k(msg_hbm, dst_hbm, fprev_hbm, out_hbm, idx_v, planes_v, rows_v,
          acc_sh, isem0, isem1, psem0, psem1):
        sid = lax.axis_index("s")
        isems = [isem0, isem1]
        psems = [psem0, psem1]

        @pl.when(sid == 0)
        def _():
            pltpu.sync_copy(fprev_hbm, acc_sh)

        plsc.subcore_barrier()
        base = sid * per
        lanes = lax.iota(_I32, 16)

        def start(i, b):
            off = base + i * ch
            pltpu.make_async_copy(
                dst_hbm.at[pl.ds(off, ch)], idx_v.at[b], isems[b]
            ).start()
            pltpu.make_async_copy(
                msg_hbm.at[:, pl.ds(off, ch)], planes_v.at[b], psems[b]
            ).start()

        def finish(i, b):
            off = base + i * ch
            pltpu.make_async_copy(
                dst_hbm.at[pl.ds(off, ch)], idx_v.at[b], isems[b]
            ).wait()
            pltpu.make_async_copy(
                msg_hbm.at[:, pl.ds(off, ch)], planes_v.at[b], psems[b]
            ).wait()

            def group(g, carry2):
                rows = g * 16 + lanes
                for f in range(8):
                    vals = planes_v[b, f, pl.ds(g * 16, 16)]
                    plsc.store_scatter(
                        rows_v, [rows, jnp.full((16,), f, _I32)], vals
                    )
                return carry2

            lax.fori_loop(0, ch // 16, group, 0)
            pltpu.sync_copy(rows_v, acc_sh.at[idx_v.at[b]], add=True)

        start(0, 0)

        def body(j, carry):
            for b in range(2):
                i = j * 2 + b

                @pl.when(i + 1 < nch)
                def _():
                    start(i + 1, 1 - b)

                finish(i, b)
            return carry

        lax.fori_loop(0, nch // 2, body, 0)
        plsc.subcore_barrier()

        @pl.when(sid == 0)
        def _():
            pltpu.sync_copy(acc_sh, out_hbm)

    return k(msg_t, dst, f_prev)


def _prelu(h, a):
    return jnp.where(h >= 0.0, h, a * h)


def _mlp_body(first, vi_ref, vj_ref, *rest):
    if first:
        (ea_ref, wa_ref, wc_ref, b0_ref, w1_ref, w2_ref, w3_ref,
         wl8_ref, alpha_ref, out_ref) = rest
        h = jnp.dot(wa_ref[...], vi_ref[...] * vj_ref[...]) + jnp.dot(
            wc_ref[...], ea_ref[...]
        )
    else:
        (fj_ref, ea_ref, wa_ref, wb_ref, wf_ref, wc_ref, b0_ref,
         w1_ref, w2_ref, w3_ref, wl8_ref, alpha_ref, out_ref) = rest
        h = (
            jnp.dot(wa_ref[...], vi_ref[...])
            + jnp.dot(wb_ref[...], vj_ref[...])
            + jnp.dot(wf_ref[...], fj_ref[...])
            + jnp.dot(wc_ref[...], ea_ref[...])
        )
    h = _prelu(h + b0_ref[...], alpha_ref[0, 0])
    h = _prelu(jnp.dot(w1_ref[...], h), alpha_ref[0, 1])
    h = _prelu(jnp.dot(w2_ref[...], h), alpha_ref[0, 2])
    h = _prelu(jnp.dot(w3_ref[...], h), alpha_ref[0, 3])
    t = jnp.dot(wl8_ref[...], h) + alpha_ref[0, 4]
    eb = t.shape[1]
    dirc = jnp.concatenate(
        [
            ea_ref[4:7, :],
            jnp.ones((1, eb), _F32),
            jnp.zeros((4, eb), _F32),
        ],
        axis=0,
    )
    out_ref[...] = t * dirc


def _tc_mlp(first, vi, vj, fj, ea_t7, wa, wb, wf, wc, b0, ws, wl8, alphas):
    e = vi.shape[1]
    eb = 2560
    assert e % eb == 0
    grid = (e // eb,)

    edge_spec = pl.BlockSpec((8, eb), lambda i: (0, i))
    ea_spec = pl.BlockSpec((7, eb), lambda i: (0, i))
    w_in_spec = pl.BlockSpec((128, 8), lambda i: (0, 0))
    w_in7_spec = pl.BlockSpec((128, 7), lambda i: (0, 0))
    w_hid_spec = pl.BlockSpec((128, 128), lambda i: (0, 0))
    tail_specs = [
        pl.BlockSpec((128, 1), lambda i: (0, 0)),
        w_hid_spec,
        w_hid_spec,
        w_hid_spec,
        pl.BlockSpec((8, 128), lambda i: (0, 0)),
        pl.BlockSpec(memory_space=pltpu.MemorySpace.SMEM),
    ]
    if first:
        specs = [edge_spec] * 2 + [ea_spec, w_in_spec, w_in7_spec] + tail_specs
        args = (vi, vj, ea_t7, wa, wc, b0, ws[0], ws[1], ws[2], wl8, alphas)
    else:
        specs = ([edge_spec] * 3 + [ea_spec] + [w_in_spec] * 3
                 + [w_in7_spec] + tail_specs)
        args = (vi, vj, fj, ea_t7, wa, wb, wf, wc, b0, ws[0], ws[1],
                ws[2], wl8, alphas)

    return pl.pallas_call(
        functools.partial(_mlp_body, first),
        grid=grid,
        in_specs=specs,
        out_specs=edge_spec,
        out_shape=jax.ShapeDtypeStruct((8, e), _F32),
    )(*args)


def _prep_weights(p, first):
    w0 = p["W0"]
    z = jnp.zeros((128, 8), _F32)
    z7 = jnp.zeros((128, 7), _F32)
    if first:
        wa = z.at[:, 0:3].set(w0[0:3].T)
        wb = z
        wf = z
        wc = z7.at[:, 0:4].set(w0[3:7].T)
    else:
        wa = z.at[:, 0:3].set(w0[0:3].T)
        wb = z.at[:, 0:3].set(w0[3:6].T)
        wf = z.at[:, 0:4].set(w0[7:11].T)
        wc = z7.at[:, 3:4].set(w0[6:7].T)
    b0 = p["b0"].reshape(128, 1)
    ws = tuple(w.T for w in p["Ws"])
    wl8 = jnp.broadcast_to(p["Wl"].T, (8, 128))
    scal = jnp.stack(
        [
            p["a0"],
            p["alphas"][0],
            p["alphas"][1],
            p["alphas"][2],
            p["bl"][0],
            jnp.float32(0.0),
            jnp.float32(0.0),
            jnp.float32(0.0),
        ]
    ).reshape(1, 8)
    return wa, wb, wf, wc, b0, ws, wl8, scal


def kernel(x, edge_index, edge_attr, params):
    n = x.shape[0]
    e = edge_index.shape[1]
    src = edge_index[0].astype(_I32)
    dst = edge_index[1].astype(_I32)

    v8 = jnp.pad(x.astype(_F32), ((0, 0), (0, 5)))
    ea_t7 = jnp.transpose(edge_attr.astype(_F32))

    vi, vj = _sc_gather_planes(v8, (dst, src), 3)

    f = jnp.zeros((n, 8), _F32)
    fj = None
    for li, name in enumerate(("mlp0", "mlp1", "mlp2")):
        first = li == 0
        wa, wb, wf, wc, b0, ws, wl8, alphas = _prep_weights(
            params[name], first
        )
        if not first:
            (fj,) = _sc_gather_planes(f, (src,), 4)
        msg_t = _tc_mlp(first, vi, vj, fj, ea_t7, wa, wb, wf, wc, b0,
                        ws, wl8, alphas)
        f = _sc_scatter_add(msg_t, dst, f)

    return f[:, :3]


# speedup vs baseline: 6.8380x; 1.0280x over previous
import functools

import jax
import jax.numpy as jnp
from jax import lax
from jax.experimental import pallas as pl
from jax.experimental.pallas import tpu as pltpu
from jax.experimental.pallas import tpu_sc as plsc

_F32 = jnp.float32
_I32 = jnp.int32
_SC_PARAMS = pltpu.CompilerParams(
    use_tc_tiling_on_sc=False, needs_layout_passes=False
)


def _sc_gather_planes(table, idxs, n_feat):
    n, d = table.shape
    e = idxs[0].shape[0]
    nw = 32
    per = e // nw
    ch = 1000
    assert e % nw == 0 and per % ch == 0 and d == 8

    mesh = plsc.VectorSubcoreMesh(core_axis_name="c", subcore_axis_name="s")
    n_out = len(idxs)

    nch = per // ch
    assert (nch * len(idxs)) % 2 == 0

    @functools.partial(
        pl.kernel,
        mesh=mesh,
        out_type=tuple(
            jax.ShapeDtypeStruct((8, e), _F32) for _ in range(n_out)
        ),
        scratch_types=[
            pltpu.VMEM((n, d), _F32),
            pltpu.VMEM((2, ch), _I32),
            pltpu.VMEM((2, 8, ch), _F32),
            pltpu.SemaphoreType.DMA,
            pltpu.SemaphoreType.DMA,
            pltpu.SemaphoreType.DMA,
            pltpu.SemaphoreType.DMA,
        ],
        compiler_params=_SC_PARAMS,
    )
    def k(table_hbm, *rest):
        idx_hbms = rest[:n_out]
        out_hbms = rest[n_out:2 * n_out]
        tab_v, idx_v, planes_v = rest[2 * n_out:2 * n_out + 3]
        isems = list(rest[2 * n_out + 3:2 * n_out + 5])
        osems = list(rest[2 * n_out + 5:2 * n_out + 7])
        wid = lax.axis_index("s") * 2 + lax.axis_index("c")
        base = wid * per
        pltpu.sync_copy(table_hbm, tab_v)
        zeros16 = jnp.zeros((16,), _F32)
        nc_tot = n_out * nch

        def idx_start(c, b):
            l = c // nch
            off = base + (c % nch) * ch
            for ll, idx_hbm in enumerate(idx_hbms):
                @pl.when(l == ll)
                def _(idx_hbm=idx_hbm):
                    pltpu.make_async_copy(
                        idx_hbm.at[pl.ds(off, ch)], idx_v.at[b], isems[b]
                    ).start()

        def chunk_work(c, b):
            l = c // nch
            off = base + (c % nch) * ch
            for ll, idx_hbm in enumerate(idx_hbms):
                @pl.when(l == ll)
                def _(idx_hbm=idx_hbm):
                    pltpu.make_async_copy(
                        idx_hbm.at[pl.ds(off, ch)], idx_v.at[b], isems[b]
                    ).wait()

            def group(g, carry):
                nodes = idx_v[b, pl.ds(g * 16, 16)]
                for f in range(8):
                    if f < n_feat:
                        vals = plsc.load_gather(
                            tab_v, [nodes, jnp.full((16,), f, _I32)]
                        )
                    else:
                        vals = zeros16
                    planes_v[b, f, pl.ds(g * 16, 16)] = vals
                return carry

            lax.fori_loop(0, ch // 16, group, 0)
            for ll, out_hbm in enumerate(out_hbms):
                @pl.when(l == ll)
                def _(out_hbm=out_hbm):
                    pltpu.make_async_copy(
                        planes_v.at[b], out_hbm.at[:, pl.ds(off, ch)],
                        osems[b],
                    ).start()

        def out_wait(c, b):
            l = c // nch
            off = base + (c % nch) * ch
            for ll, out_hbm in enumerate(out_hbms):
                @pl.when(l == ll)
                def _(out_hbm=out_hbm):
                    pltpu.make_async_copy(
                        planes_v.at[b], out_hbm.at[:, pl.ds(off, ch)],
                        osems[b],
                    ).wait()

        idx_start(0, 0)

        def body(j, carry):
            for b in range(2):
                c = j * 2 + b

                @pl.when(c + 1 < nc_tot)
                def _():
                    idx_start(c + 1, 1 - b)

                @pl.when(c >= 2)
                def _():
                    out_wait(c - 2, b)

                chunk_work(c, b)
            return carry

        lax.fori_loop(0, nc_tot // 2, body, 0)
        out_wait(nc_tot - 2, 0)
        out_wait(nc_tot - 1, 1)

    return k(table, *idxs)


def _sc_scatter_add(msg_t, dst, f_prev):
    d, e = msg_t.shape
    n = f_prev.shape[0]
    nt = 16
    per = e // nt
    ch = 2000
    assert e % nt == 0 and per % ch == 0 and d == 8

    mesh = plsc.VectorSubcoreMesh(
        core_axis_name="c", subcore_axis_name="s", num_cores=1
    )

    nch = per // ch
    assert nch % 2 == 0

    @functools.partial(
        pl.kernel,
        mesh=mesh,
        out_type=jax.ShapeDtypeStruct((n, d), _F32),
        scratch_types=[
            pltpu.VMEM((2, ch), _I32),
            pltpu.VMEM((2, 8, ch), _F32),
            pltpu.VMEM((ch, 8), _F32),
            pltpu.VMEM_SHARED((n, d), _F32),
            pltpu.SemaphoreType.DMA,
            pltpu.SemaphoreType.DMA,
            pltpu.SemaphoreType.DMA,
            pltpu.SemaphoreType.DMA,
        ],
        compiler_params=_SC_PARAMS,
    )
    def k(msg_hbm, dst_hbm, fprev_hbm, out_hbm, idx_v, planes_v, rows_v,
          acc_sh, isem0, isem1, psem0, psem1):
        sid = lax.axis_index("s")
        isems = [isem0, isem1]
        psems = [psem0, psem1]

        @pl.when(sid == 0)
        def _():
            pltpu.sync_copy(fprev_hbm, acc_sh)

        plsc.subcore_barrier()
        base = sid * per
        lanes = lax.iota(_I32, 16)

        def start(i, b):
            off = base + i * ch
            pltpu.make_async_copy(
                dst_hbm.at[pl.ds(off, ch)], idx_v.at[b], isems[b]
            ).start()
            pltpu.make_async_copy(
                msg_hbm.at[:, pl.ds(off, ch)], planes_v.at[b], psems[b]
            ).start()

        def finish(i, b):
            off = base + i * ch
            pltpu.make_async_copy(
                dst_hbm.at[pl.ds(off, ch)], idx_v.at[b], isems[b]
            ).wait()
            pltpu.make_async_copy(
                msg_hbm.at[:, pl.ds(off, ch)], planes_v.at[b], psems[b]
            ).wait()

            def group(g, carry2):
                rows = g * 16 + lanes
                for f in range(8):
                    vals = planes_v[b, f, pl.ds(g * 16, 16)]
                    plsc.store_scatter(
                        rows_v, [rows, jnp.full((16,), f, _I32)], vals
                    )
                return carry2

            lax.fori_loop(0, ch // 16, group, 0)
            pltpu.sync_copy(rows_v, acc_sh.at[idx_v.at[b]], add=True)

        start(0, 0)

        def body(j, carry):
            for b in range(2):
                i = j * 2 + b

                @pl.when(i + 1 < nch)
                def _():
                    start(i + 1, 1 - b)

                finish(i, b)
            return carry

        lax.fori_loop(0, nch // 2, body, 0)
        plsc.subcore_barrier()

        @pl.when(sid == 0)
        def _():
            pltpu.sync_copy(acc_sh, out_hbm)

    return k(msg_t, dst, f_prev)


def _prelu(h, a):
    return jnp.where(h >= 0.0, h, a * h)


def _mlp_body(first, vi_ref, vj_ref, *rest):
    if first:
        (ea_ref, wa_ref, wc_ref, b0_ref, w1_ref, w2_ref, w3_ref,
         wl8_ref, alpha_ref, out_ref) = rest
        h = jnp.dot(wa_ref[...], vi_ref[...] * vj_ref[...]) + jnp.dot(
            wc_ref[...], ea_ref[...]
        )
    else:
        (fj_ref, ea_ref, wa_ref, wb_ref, wf_ref, wc_ref, b0_ref,
         w1_ref, w2_ref, w3_ref, wl8_ref, alpha_ref, out_ref) = rest
        h = (
            jnp.dot(wa_ref[...], vi_ref[...])
            + jnp.dot(wb_ref[...], vj_ref[...])
            + jnp.dot(wf_ref[...], fj_ref[...])
            + jnp.dot(wc_ref[...], ea_ref[...])
        )
    h = _prelu(h + b0_ref[...], alpha_ref[0, 0])
    h = _prelu(jnp.dot(w1_ref[...], h), alpha_ref[0, 1])
    h = _prelu(jnp.dot(w2_ref[...], h), alpha_ref[0, 2])
    h = _prelu(jnp.dot(w3_ref[...], h), alpha_ref[0, 3])
    t = jnp.dot(wl8_ref[...], h) + alpha_ref[0, 4]
    eb = t.shape[1]
    dirc = jnp.concatenate(
        [
            ea_ref[4:7, :],
            jnp.ones((1, eb), _F32),
            jnp.zeros((4, eb), _F32),
        ],
        axis=0,
    )
    out_ref[...] = t * dirc


def _tc_mlp(first, vi, vj, fj, ea_t7, wa, wb, wf, wc, b0, ws, wl8, alphas):
    e = vi.shape[1]
    eb = 2560
    assert e % eb == 0
    grid = (e // eb,)

    edge_spec = pl.BlockSpec((8, eb), lambda i: (0, i))
    ea_spec = pl.BlockSpec((7, eb), lambda i: (0, i))
    w_in_spec = pl.BlockSpec((128, 8), lambda i: (0, 0))
    w_in7_spec = pl.BlockSpec((128, 7), lambda i: (0, 0))
    w_hid_spec = pl.BlockSpec((128, 128), lambda i: (0, 0))
    tail_specs = [
        pl.BlockSpec((128, 1), lambda i: (0, 0)),
        w_hid_spec,
        w_hid_spec,
        w_hid_spec,
        pl.BlockSpec((8, 128), lambda i: (0, 0)),
        pl.BlockSpec(memory_space=pltpu.MemorySpace.SMEM),
    ]
    if first:
        specs = [edge_spec] * 2 + [ea_spec, w_in_spec, w_in7_spec] + tail_specs
        args = (vi, vj, ea_t7, wa, wc, b0, ws[0], ws[1], ws[2], wl8, alphas)
    else:
        specs = ([edge_spec] * 3 + [ea_spec] + [w_in_spec] * 3
                 + [w_in7_spec] + tail_specs)
        args = (vi, vj, fj, ea_t7, wa, wb, wf, wc, b0, ws[0], ws[1],
                ws[2], wl8, alphas)

    return pl.pallas_call(
        functools.partial(_mlp_body, first),
        grid=grid,
        in_specs=specs,
        out_specs=edge_spec,
        out_shape=jax.ShapeDtypeStruct((8, e), _F32),
    )(*args)


def _prep_weights(p, first):
    w0 = p["W0"]
    z = jnp.zeros((128, 8), _F32)
    z7 = jnp.zeros((128, 7), _F32)
    if first:
        wa = z.at[:, 0:3].set(w0[0:3].T)
        wb = z
        wf = z
        wc = z7.at[:, 0:4].set(w0[3:7].T)
    else:
        wa = z.at[:, 0:3].set(w0[0:3].T)
        wb = z.at[:, 0:3].set(w0[3:6].T)
        wf = z.at[:, 0:4].set(w0[7:11].T)
        wc = z7.at[:, 3:4].set(w0[6:7].T)
    b0 = p["b0"].reshape(128, 1)
    ws = tuple(w.T for w in p["Ws"])
    wl8 = jnp.broadcast_to(p["Wl"].T, (8, 128))
    scal = jnp.stack(
        [
            p["a0"],
            p["alphas"][0],
            p["alphas"][1],
            p["alphas"][2],
            p["bl"][0],
            jnp.float32(0.0),
            jnp.float32(0.0),
            jnp.float32(0.0),
        ]
    ).reshape(1, 8)
    return wa, wb, wf, wc, b0, ws, wl8, scal


def kernel(x, edge_index, edge_attr, params):
    n = x.shape[0]
    e = edge_index.shape[1]
    src = edge_index[0].astype(_I32)
    dst = edge_index[1].astype(_I32)

    v8 = jnp.pad(x.astype(_F32), ((0, 0), (0, 5)))
    ea_t7 = jnp.transpose(edge_attr.astype(_F32))

    vi, vj = _sc_gather_planes(v8, (dst, src), 3)

    f = jnp.zeros((n, 8), _F32)
    fj = None
    for li, name in enumerate(("mlp0", "mlp1", "mlp2")):
        first = li == 0
        wa, wb, wf, wc, b0, ws, wl8, alphas = _prep_weights(
            params[name], first
        )
        if not first:
            (fj,) = _sc_gather_planes(f, (src,), 4)
        msg_t = _tc_mlp(first, vi, vj, fj, ea_t7, wa, wb, wf, wc, b0,
                        ws, wl8, alphas)
        f = _sc_scatter_add(msg_t, dst, f)

    return f[:, :3]
